# baseline (speedup 1.0000x reference)
import functools

import jax
import jax.numpy as jnp
from jax import lax
from jax.experimental import pallas as pl
from jax.experimental.pallas import tpu as pltpu
from jax.experimental.pallas import tpu_sc as plsc

_VOCAB = 100000
_D = 128
_B = 4096
_F = 26
_N = _B * _F
_NC = 2
_NS = 16
_NW = _NC * _NS
_PER_W = _N // _NW
_CHUNK = 128
_NCHUNK = _PER_W // _CHUNK
_STEP = 2 * _CHUNK
_NSTEP = _NCHUNK // 2


def _make_gather():
  mesh = plsc.VectorSubcoreMesh(core_axis_name="c", subcore_axis_name="s")

  @functools.partial(
      pl.kernel,
      mesh=mesh,
      out_type=jax.ShapeDtypeStruct((_N, _D), jnp.float32),
      scratch_types=[
          pltpu.VMEM((_NCHUNK, _CHUNK), jnp.int32),
          pltpu.VMEM((_STEP, _D), jnp.float32),
          pltpu.VMEM((_STEP, _D), jnp.float32),
          pltpu.VMEM((_STEP, _D), jnp.float32),
          pltpu.SemaphoreType.DMA,
          pltpu.SemaphoreType.DMA,
          pltpu.SemaphoreType.DMA,
          pltpu.SemaphoreType.DMA,
          pltpu.SemaphoreType.DMA,
          pltpu.SemaphoreType.DMA,
      ],
  )
  def gather_kernel(table_hbm, idx_hbm, out_hbm, idx_v, buf_a, buf_b, buf_c,
                    ga, gb, gc, wa, wsb, wc):
    wid = lax.axis_index("s") * _NC + lax.axis_index("c")
    base = wid * _PER_W
    pltpu.sync_copy(idx_hbm.at[wid], idx_v)

    def g_start(step, buf, sem):
      c0 = 2 * step
      pltpu.async_copy(
          table_hbm.at[idx_v.at[c0]], buf.at[pl.ds(0, _CHUNK)], sem)
      pltpu.async_copy(
          table_hbm.at[idx_v.at[c0 + 1]], buf.at[pl.ds(_CHUNK, _CHUNK)], sem)

    def g_wait(step, buf, sem):
      c0 = 2 * step
      pltpu.make_async_copy(
          table_hbm.at[idx_v.at[c0]], buf.at[pl.ds(0, _CHUNK)], sem).wait()
      pltpu.make_async_copy(
          table_hbm.at[idx_v.at[c0 + 1]], buf.at[pl.ds(_CHUNK, _CHUNK)],
          sem).wait()

    def w_start(step, buf, sem):
      pltpu.async_copy(
          buf, out_hbm.at[pl.ds(base + step * _STEP, _STEP)], sem)

    def w_wait(step, buf, sem):
      pltpu.make_async_copy(
          buf, out_hbm.at[pl.ds(base + step * _STEP, _STEP)], sem).wait()

    sets = ((buf_a, ga, wa), (buf_b, gb, wsb), (buf_c, gc, wc))

    def stage(j, cur, nxt, last):
      g_wait(j, cur[0], cur[1])
      w_start(j, cur[0], cur[2])
      w_wait(j - 1, nxt[0], nxt[2])
      if not last:
        g_start(j + 2, nxt[0], nxt[1])

    g_start(0, sets[0][0], sets[0][1])
    g_start(1, sets[1][0], sets[1][1])
    g_wait(0, sets[0][0], sets[0][1])
    w_start(0, sets[0][0], sets[0][2])
    g_start(2, sets[2][0], sets[2][1])
    stage(1, sets[1], sets[0], False)

    def body(i, carry):
      j0 = 2 + 3 * i
      stage(j0, sets[2], sets[1], False)
      stage(j0 + 1, sets[0], sets[2], False)
      stage(j0 + 2, sets[1], sets[0], False)
      return carry

    lax.fori_loop(0, (_NSTEP - 4) // 3, body, 0)

    stage(11, sets[2], sets[1], True)
    stage(12, sets[0], sets[2], True)
    w_wait(12, sets[0][0], sets[0][2])

  return gather_kernel


_gather = _make_gather()


def kernel(input_batch, table):
  idx3 = input_batch.T.reshape(_NW, _NCHUNK, _CHUNK)
  out = _gather(table, idx3)
  return out.reshape(_F, _B, _D).transpose(1, 0, 2)
